# Initial kernel scaffold; baseline (speedup 1.0000x reference)
import jax
import jax.numpy as jnp
from jax.experimental import pallas as pl


def kernel(hidden_states, gate_w, gate_ws, up_ws, down_ws, shared_gate_w, shared_up_w, shared_down_w):
    raise NotImplementedError("write your pallas kernel here")


# speedup vs baseline: 1.1794x; 1.1794x over previous
import functools

import jax
import jax.numpy as jnp
from jax.experimental import pallas as pl
from jax.experimental.pallas import tpu as pltpu

H = 2048
E = 8
F = 512
TOPK = 2
NGROUP = 4
GSZ = E // NGROUP
TOPKG = 2
SF = 1024
T = 2048


def _router_body(x_ref, gw_ref, cmb_ref, xbf_ref):
    x = x_ref[...]
    gw = gw_ref[...]
    logits = jax.lax.dot_general(
        x, gw, (((1,), (1,)), ((), ())), preferred_element_type=jnp.float32)
    m = jnp.max(logits, axis=-1, keepdims=True)
    ex = jnp.exp(logits - m)
    scores = ex / jnp.sum(ex, axis=-1, keepdims=True)

    cols = [scores[:, i:i + 1] for i in range(E)]
    gexp_cols = []
    for g in range(NGROUP):
        gmax = jnp.maximum(cols[2 * g], cols[2 * g + 1])
        gexp_cols.append(gmax)
    grank = []
    for g in range(NGROUP):
        r = jnp.zeros_like(gexp_cols[0])
        for j in range(NGROUP):
            if j == g:
                continue
            gt = gexp_cols[j] > gexp_cols[g]
            eq = (gexp_cols[j] == gexp_cols[g]) & (j < g)
            r = r + jnp.where(gt | eq, 1.0, 0.0)
        grank.append(r)
    masked_cols = []
    for e in range(E):
        keep = grank[e // GSZ] < float(TOPKG)
        masked_cols.append(jnp.where(keep, cols[e], 0.0))
    combine_cols = []
    for e in range(E):
        r = jnp.zeros_like(masked_cols[0])
        for j in range(E):
            if j == e:
                continue
            gt = masked_cols[j] > masked_cols[e]
            eq = (masked_cols[j] == masked_cols[e]) & (j < e)
            r = r + jnp.where(gt | eq, 1.0, 0.0)
        combine_cols.append(jnp.where(r < float(TOPK), masked_cols[e], 0.0))
    cmb_ref[...] = jnp.concatenate(combine_cols, axis=1)
    xbf_ref[...] = x.astype(jnp.bfloat16)


def _router(x, gate_w):
    return pl.pallas_call(
        _router_body,
        out_shape=(
            jax.ShapeDtypeStruct((T, E), jnp.float32),
            jax.ShapeDtypeStruct((T, H), jnp.bfloat16),
        ),
    )(x, gate_w)


FC = 2
FB = F // FC


def _routed_body(xbf_ref, wg_ref, wu_ref, wd_ref, cmb_ref, out_ref):
    step = pl.program_id(0)
    e = step // FC

    @pl.when(step == 0)
    def _():
        out_ref[...] = jnp.zeros_like(out_ref)

    xb = xbf_ref[...]
    wg = wg_ref[0].astype(jnp.bfloat16)
    wu = wu_ref[0].astype(jnp.bfloat16)
    wd = wd_ref[0].astype(jnp.bfloat16)
    g = jax.lax.dot_general(xb, wg, (((1,), (1,)), ((), ())),
                            preferred_element_type=jnp.float32)
    u = jax.lax.dot_general(xb, wu, (((1,), (1,)), ((), ())),
                            preferred_element_type=jnp.float32)
    h = (g * jax.nn.sigmoid(g) * u).astype(jnp.bfloat16)
    y = jax.lax.dot_general(h, wd, (((1,), (1,)), ((), ())),
                            preferred_element_type=jnp.float32)
    lane = jax.lax.broadcasted_iota(jnp.int32, (T, E), 1)
    c = jnp.sum(jnp.where(lane == e, cmb_ref[...], 0.0), axis=1,
                keepdims=True)
    out_ref[...] += (y * c).astype(out_ref.dtype)


def _routed(xbf, gate_ws, up_ws, down_ws, cmb):
    return pl.pallas_call(
        _routed_body,
        grid=(E * FC,),
        in_specs=[
            pl.BlockSpec((T, H), lambda i: (0, 0)),
            pl.BlockSpec((1, FB, H), lambda i: (i // FC, i % FC, 0)),
            pl.BlockSpec((1, FB, H), lambda i: (i // FC, i % FC, 0)),
            pl.BlockSpec((1, H, FB), lambda i: (i // FC, 0, i % FC)),
            pl.BlockSpec((T, E), lambda i: (0, 0)),
        ],
        out_specs=pl.BlockSpec((T, H), lambda i: (0, 0)),
        out_shape=jax.ShapeDtypeStruct((T, H), jnp.bfloat16),
    )(xbf, gate_ws, up_ws, down_ws, cmb)


SFC = 4
SFB = SF // SFC


def _shared_body(xbf_ref, wg_ref, wu_ref, wd_ref, routed_ref, out_ref):
    step = pl.program_id(0)

    @pl.when(step == 0)
    def _():
        out_ref[...] = routed_ref[...].astype(jnp.float32)

    xb = xbf_ref[...]
    wg = wg_ref[...].astype(jnp.bfloat16)
    wu = wu_ref[...].astype(jnp.bfloat16)
    wd = wd_ref[...].astype(jnp.bfloat16)
    g = jax.lax.dot_general(xb, wg, (((1,), (1,)), ((), ())),
                            preferred_element_type=jnp.float32)
    u = jax.lax.dot_general(xb, wu, (((1,), (1,)), ((), ())),
                            preferred_element_type=jnp.float32)
    h = (g * jax.nn.sigmoid(g) * u).astype(jnp.bfloat16)
    out_ref[...] += jax.lax.dot_general(h, wd, (((1,), (1,)), ((), ())),
                                        preferred_element_type=jnp.float32)


def _shared(xbf, sg_w, su_w, sd_w, routed):
    return pl.pallas_call(
        _shared_body,
        grid=(SFC,),
        in_specs=[
            pl.BlockSpec((T, H), lambda i: (0, 0)),
            pl.BlockSpec((SFB, H), lambda i: (i, 0)),
            pl.BlockSpec((SFB, H), lambda i: (i, 0)),
            pl.BlockSpec((H, SFB), lambda i: (0, i)),
            pl.BlockSpec((T, H), lambda i: (0, 0)),
        ],
        out_specs=pl.BlockSpec((T, H), lambda i: (0, 0)),
        out_shape=jax.ShapeDtypeStruct((T, H), jnp.float32),
    )(xbf, sg_w, su_w, sd_w, routed)


def kernel(hidden_states, gate_w, gate_ws, up_ws, down_ws,
           shared_gate_w, shared_up_w, shared_down_w):
    cmb, xbf = _router(hidden_states, gate_w)
    routed = _routed(xbf, gate_ws, up_ws, down_ws, cmb)
    return _shared(xbf, shared_gate_w, shared_up_w, shared_down_w, routed)
